# baseline (speedup 1.0000x reference)
import functools

import jax
import jax.numpy as jnp
from jax import lax
from jax.experimental import pallas as pl
from jax.experimental.pallas import tpu as pltpu
from jax.experimental.pallas import tpu_sc as plsc

B = 16384
H = 50
D = 32
NC = 2
NS = 16
NW = NC * NS
BPW = B // NW

CH = 32
NCH = BPW // CH
GRP = 80
NG = CH * H // GRP


def _pool_body(idx_hbm, table_hbm, b_hbm, out_hbm, idx_v, rows_v, out_v, b_v,
               sems):
    wid = lax.axis_index("s") * NC + lax.axis_index("c")
    pltpu.sync_copy(b_hbm, b_v)
    b_lo = b_v[pl.ds(0, 16)]
    b_hi = b_v[pl.ds(16, 16)]

    def fire(c, buf):
        ibase = pl.multiple_of((wid * BPW + c * CH) * H, CH * H)
        pltpu.sync_copy(idx_hbm.at[pl.ds(ibase, CH * H)], idx_v.at[buf])
        for g in range(NG):
            pltpu.async_copy(
                table_hbm.at[idx_v.at[buf, pl.ds(g * GRP, GRP)]],
                rows_v.at[buf, pl.ds(g * GRP, GRP)],
                sems.at[buf],
            )

    def drain(buf):
        for g in range(NG):
            pltpu.make_async_copy(
                table_hbm.at[idx_v.at[buf, pl.ds(g * GRP, GRP)]],
                rows_v.at[buf, pl.ds(g * GRP, GRP)],
                sems.at[buf],
            ).wait()

    def process(c, buf):
        drain(buf)

        def row_body(r, rcarry):
            base = r * H
            lo, hi = plsc.unpack(
                rows_v[buf, base, :], format=plsc.PackFormat.INTERLEAVED)
            for l in range(1, H):
                dlo, dhi = plsc.unpack(
                    rows_v[buf, base + l, :], format=plsc.PackFormat.INTERLEAVED)
                lo = lo + dlo
                hi = hi + dhi
            zero = jnp.zeros((16,), jnp.float32)
            out_v[r, pl.ds(0, 16)] = jnp.maximum(lo + b_lo, zero)
            out_v[r, pl.ds(16, 16)] = jnp.maximum(hi + b_hi, zero)
            return rcarry

        lax.fori_loop(0, CH, row_body, 0)
        row0 = pl.multiple_of(wid * BPW + c * CH, CH)
        pltpu.sync_copy(out_v, out_hbm.at[pl.ds(row0, CH)])

    fire(0, 0)

    def pair_body(i, carry):
        c0 = i * 2
        fire(c0 + 1, 1)
        process(c0, 0)

        @pl.when(c0 + 2 < NCH)
        def _():
            fire(c0 + 2, 0)

        process(c0 + 1, 1)
        return carry

    lax.fori_loop(0, NCH // 2, pair_body, 0)


_pool = functools.partial(
    pl.kernel,
    out_type=jax.ShapeDtypeStruct((B, D), jnp.float32),
    mesh=plsc.VectorSubcoreMesh(core_axis_name="c", subcore_axis_name="s"),
    scratch_types=[
        pltpu.VMEM((2, CH * H), jnp.int32),
        pltpu.VMEM((2, CH * H, D), jnp.bfloat16),
        pltpu.VMEM((CH, D), jnp.float32),
        pltpu.VMEM((D,), jnp.float32),
        pltpu.SemaphoreType.DMA((2,)),
    ],
    compiler_params=pltpu.CompilerParams(
        use_tc_tiling_on_sc=False, needs_layout_passes=False),
)(_pool_body)


TPC = 32768
TPQ = TPC // 4
TPQ_SHIFT = TPQ.bit_length() - 1


def _tp_body(t_ref, w_ref, o_ref):
    x = t_ref[...]
    w = w_ref[...]
    xcat = jnp.concatenate(
        [x[:, a * TPQ:(a + 1) * TPQ] for a in range(4)], axis=0)
    wv = jnp.concatenate([w, w, w, w], axis=0)
    wbd = jnp.concatenate([wv, wv, wv, wv], axis=1)
    row = lax.broadcasted_iota(jnp.int32, (4 * D, 4 * D), 0)
    col = lax.broadcasted_iota(jnp.int32, (4 * D, 4 * D), 1)
    wbd = jnp.where(row // D == col // D, wbd, 0.0)
    acc = lax.dot_general(
        xcat, wbd, (((0,), (0,)), ((), ())),
        preferred_element_type=jnp.float32)
    o_ref[...] = acc.astype(jnp.bfloat16)


def _transpose_table(table_t, W):
    V = table_t.shape[1]
    nblk = pl.cdiv(V, TPC)
    out = pl.pallas_call(
        _tp_body,
        grid=(nblk,),
        in_specs=[pl.BlockSpec((D, TPC), lambda i: (0, i)),
                  pl.BlockSpec((D, D), lambda i: (0, 0))],
        out_specs=pl.BlockSpec((TPQ, 4 * D), lambda i: (i, 0)),
        out_shape=jax.ShapeDtypeStruct((nblk * TPQ, 4 * D), jnp.bfloat16),
    )(table_t, W)
    return out.reshape(nblk * TPC, D)


def _permute_indices(i):
    return (i & ~(TPC - 1)) + ((i & (TPQ - 1)) << 2) + ((i & (TPC - 1)) >> TPQ_SHIFT)


def kernel(indices, table, W, b):
    idx_flat = _permute_indices(indices.reshape(-1).astype(jnp.int32))
    wperm = jnp.arange(D).reshape(2, D // 2).T.reshape(-1)
    table_w = _transpose_table(table.T, W[:, wperm])
    return _pool(idx_flat, table_w, b)


# speedup vs baseline: 2.0396x; 2.0396x over previous
import functools

import jax
import jax.numpy as jnp
from jax import lax
from jax.experimental import pallas as pl
from jax.experimental.pallas import tpu as pltpu
from jax.experimental.pallas import tpu_sc as plsc

B = 16384
H = 50
D = 32
NC = 2
NS = 16
NW = NC * NS
BPW = B // NW

CH = 32
NCH = BPW // CH
GRP = 80
NG = CH * H // GRP


def _pool_body(idx_hbm, table_hbm, b_hbm, out_hbm, idx_v, rows_v, out_v, b_v,
               sems):
    wid = lax.axis_index("s") * NC + lax.axis_index("c")
    pltpu.sync_copy(b_hbm, b_v)
    b_lo = b_v[pl.ds(0, 16)]
    b_hi = b_v[pl.ds(16, 16)]

    def fire(c, buf):
        ibase = pl.multiple_of((wid * BPW + c * CH) * H, CH * H)
        pltpu.sync_copy(idx_hbm.at[pl.ds(ibase, CH * H)], idx_v.at[buf])
        for g in range(NG):
            pltpu.async_copy(
                table_hbm.at[idx_v.at[buf, pl.ds(g * GRP, GRP)]],
                rows_v.at[buf, pl.ds(g * GRP, GRP)],
                sems.at[buf],
            )

    def drain(buf):
        for g in range(NG):
            pltpu.make_async_copy(
                table_hbm.at[idx_v.at[buf, pl.ds(g * GRP, GRP)]],
                rows_v.at[buf, pl.ds(g * GRP, GRP)],
                sems.at[buf],
            ).wait()

    def process(c, buf):
        drain(buf)

        def row_body(r, rcarry):
            base = r * H
            nacc = 4
            los = [rows_v[buf, base + k, pl.ds(0, 16)] for k in range(nacc)]
            his = [rows_v[buf, base + k, pl.ds(16, 16)] for k in range(nacc)]
            for l in range(nacc, H):
                k = l % nacc
                los[k] = los[k] + rows_v[buf, base + l, pl.ds(0, 16)]
                his[k] = his[k] + rows_v[buf, base + l, pl.ds(16, 16)]
            lo = (los[0] + los[1]) + (los[2] + los[3])
            hi = (his[0] + his[1]) + (his[2] + his[3])
            zero = jnp.zeros((16,), jnp.float32)
            out_v[r, pl.ds(0, 16)] = jnp.maximum(lo + b_lo, zero)
            out_v[r, pl.ds(16, 16)] = jnp.maximum(hi + b_hi, zero)
            return rcarry

        lax.fori_loop(0, CH, row_body, 0)
        row0 = pl.multiple_of(wid * BPW + c * CH, CH)
        pltpu.sync_copy(out_v, out_hbm.at[pl.ds(row0, CH)])

    fire(0, 0)

    def pair_body(i, carry):
        c0 = i * 2
        fire(c0 + 1, 1)
        process(c0, 0)

        @pl.when(c0 + 2 < NCH)
        def _():
            fire(c0 + 2, 0)

        process(c0 + 1, 1)
        return carry

    lax.fori_loop(0, NCH // 2, pair_body, 0)


_pool = functools.partial(
    pl.kernel,
    out_type=jax.ShapeDtypeStruct((B, D), jnp.float32),
    mesh=plsc.VectorSubcoreMesh(core_axis_name="c", subcore_axis_name="s"),
    scratch_types=[
        pltpu.VMEM((2, CH * H), jnp.int32),
        pltpu.VMEM((2, CH * H, D), jnp.float32),
        pltpu.VMEM((CH, D), jnp.float32),
        pltpu.VMEM((D,), jnp.float32),
        pltpu.SemaphoreType.DMA((2,)),
    ],
    compiler_params=pltpu.CompilerParams(
        use_tc_tiling_on_sc=False, needs_layout_passes=False),
)(_pool_body)


TPC = 32768
TPQ = TPC // 4
TPQ_SHIFT = TPQ.bit_length() - 1


def _tp_body(t_ref, w_ref, o_ref):
    x = t_ref[...]
    w = w_ref[...]
    xcat = jnp.concatenate(
        [x[:, a * TPQ:(a + 1) * TPQ] for a in range(4)], axis=0)
    wv = jnp.concatenate([w, w, w, w], axis=0)
    wbd = jnp.concatenate([wv, wv, wv, wv], axis=1)
    row = lax.broadcasted_iota(jnp.int32, (4 * D, 4 * D), 0)
    col = lax.broadcasted_iota(jnp.int32, (4 * D, 4 * D), 1)
    wbd = jnp.where(row // D == col // D, wbd, 0.0)
    acc = lax.dot_general(
        xcat, wbd, (((0,), (0,)), ((), ())),
        preferred_element_type=jnp.float32)
    o_ref[...] = acc


def _transpose_table(table_t, W):
    V = table_t.shape[1]
    nblk = pl.cdiv(V, TPC)
    out = pl.pallas_call(
        _tp_body,
        grid=(nblk,),
        in_specs=[pl.BlockSpec((D, TPC), lambda i: (0, i)),
                  pl.BlockSpec((D, D), lambda i: (0, 0))],
        out_specs=pl.BlockSpec((TPQ, 4 * D), lambda i: (i, 0)),
        out_shape=jax.ShapeDtypeStruct((nblk * TPQ, 4 * D), jnp.float32),
    )(table_t, W)
    return out.reshape(nblk * TPC, D)


def _permute_indices(i):
    return (i & ~(TPC - 1)) + ((i & (TPQ - 1)) << 2) + ((i & (TPC - 1)) >> TPQ_SHIFT)


def kernel(indices, table, W, b):
    idx_flat = _permute_indices(indices.reshape(-1).astype(jnp.int32))
    table_w = _transpose_table(table.T, W)
    return _pool(idx_flat, table_w, b)


# speedup vs baseline: 2.0526x; 1.0064x over previous
import functools

import jax
import jax.numpy as jnp
from jax import lax
from jax.experimental import pallas as pl
from jax.experimental.pallas import tpu as pltpu
from jax.experimental.pallas import tpu_sc as plsc

B = 16384
H = 50
D = 32
NC = 2
NS = 16
NW = NC * NS
BPW = B // NW

CH = 32
NCH = BPW // CH
GRP = 80
NG = CH * H // GRP


def _pool_body(idx_hbm, table_hbm, b_hbm, out_hbm, idx_v, rows_v, out_v, b_v,
               sems):
    wid = lax.axis_index("s") * NC + lax.axis_index("c")
    pltpu.sync_copy(b_hbm, b_v)
    b_lo = b_v[pl.ds(0, 16)]
    b_hi = b_v[pl.ds(16, 16)]

    def fire(c, buf):
        ibase = pl.multiple_of((wid * BPW + c * CH) * H, CH * H)
        pltpu.sync_copy(idx_hbm.at[pl.ds(ibase, CH * H)], idx_v.at[buf])
        for g in range(NG):
            pltpu.async_copy(
                table_hbm.at[idx_v.at[buf, pl.ds(g * GRP, GRP)]],
                rows_v.at[buf, pl.ds(g * GRP, GRP)],
                sems.at[buf],
            )

    def drain(buf):
        for g in range(NG):
            pltpu.make_async_copy(
                table_hbm.at[idx_v.at[buf, pl.ds(g * GRP, GRP)]],
                rows_v.at[buf, pl.ds(g * GRP, GRP)],
                sems.at[buf],
            ).wait()

    def process(c, buf):
        drain(buf)

        def row_body(r, rcarry):
            base = r * H
            nacc = 4
            los = [rows_v[buf, base + k, pl.ds(0, 16)] for k in range(nacc)]
            his = [rows_v[buf, base + k, pl.ds(16, 16)] for k in range(nacc)]
            for l in range(nacc, H):
                k = l % nacc
                los[k] = los[k] + rows_v[buf, base + l, pl.ds(0, 16)]
                his[k] = his[k] + rows_v[buf, base + l, pl.ds(16, 16)]
            lo = (los[0] + los[1]) + (los[2] + los[3])
            hi = (his[0] + his[1]) + (his[2] + his[3])
            zero = jnp.zeros((16,), jnp.float32)
            out_v[r, pl.ds(0, 16)] = jnp.maximum(lo + b_lo, zero)
            out_v[r, pl.ds(16, 16)] = jnp.maximum(hi + b_hi, zero)
            return rcarry

        lax.fori_loop(0, CH, row_body, 0)
        row0 = pl.multiple_of(wid * BPW + c * CH, CH)
        pltpu.sync_copy(out_v, out_hbm.at[pl.ds(row0, CH)])

    fire(0, 0)

    def pair_body(i, carry):
        c0 = i * 2
        fire(c0 + 1, 1)
        process(c0, 0)

        @pl.when(c0 + 2 < NCH)
        def _():
            fire(c0 + 2, 0)

        process(c0 + 1, 1)
        return carry

    lax.fori_loop(0, NCH // 2, pair_body, 0)


_pool = functools.partial(
    pl.kernel,
    out_type=jax.ShapeDtypeStruct((B, D), jnp.float32),
    mesh=plsc.VectorSubcoreMesh(core_axis_name="c", subcore_axis_name="s"),
    scratch_types=[
        pltpu.VMEM((2, CH * H), jnp.int32),
        pltpu.VMEM((2, CH * H, D), jnp.float32),
        pltpu.VMEM((CH, D), jnp.float32),
        pltpu.VMEM((D,), jnp.float32),
        pltpu.SemaphoreType.DMA((2,)),
    ],
    compiler_params=pltpu.CompilerParams(
        use_tc_tiling_on_sc=False, needs_layout_passes=False),
)(_pool_body)


TPC = 65536
TPQ = TPC // 4
TPQ_SHIFT = TPQ.bit_length() - 1


def _tp_body(t_ref, w_ref, o_ref):
    x = t_ref[...]
    w = w_ref[...]
    xcat = jnp.concatenate(
        [x[:, a * TPQ:(a + 1) * TPQ] for a in range(4)], axis=0)
    wv = jnp.concatenate([w, w, w, w], axis=0)
    wbd = jnp.concatenate([wv, wv, wv, wv], axis=1)
    row = lax.broadcasted_iota(jnp.int32, (4 * D, 4 * D), 0)
    col = lax.broadcasted_iota(jnp.int32, (4 * D, 4 * D), 1)
    wbd = jnp.where(row // D == col // D, wbd, 0.0)
    acc = lax.dot_general(
        xcat, wbd, (((0,), (0,)), ((), ())),
        preferred_element_type=jnp.float32)
    o_ref[...] = acc


def _transpose_table(table_t, W):
    V = table_t.shape[1]
    nblk = pl.cdiv(V, TPC)
    out = pl.pallas_call(
        _tp_body,
        grid=(nblk,),
        in_specs=[pl.BlockSpec((D, TPC), lambda i: (0, i)),
                  pl.BlockSpec((D, D), lambda i: (0, 0))],
        out_specs=pl.BlockSpec((TPQ, 4 * D), lambda i: (i, 0)),
        out_shape=jax.ShapeDtypeStruct((nblk * TPQ, 4 * D), jnp.float32),
    )(table_t, W)
    return out.reshape(nblk * TPC, D)


def _permute_indices(i):
    return (i & ~(TPC - 1)) + ((i & (TPQ - 1)) << 2) + ((i & (TPC - 1)) >> TPQ_SHIFT)


def kernel(indices, table, W, b):
    idx_flat = _permute_indices(indices.reshape(-1).astype(jnp.int32))
    table_w = _transpose_table(table.T, W)
    return _pool(idx_flat, table_w, b)
